# baseline (speedup 1.0000x reference)
import functools

import jax
import jax.numpy as jnp
from jax import lax
from jax.experimental import pallas as pl
from jax.experimental.pallas import tpu as pltpu
from jax.experimental.pallas import tpu_sc as plsc

N = 10000
E = 320000
G = 64

NC = 2
NS = 16
NW = NC * NS

NP = 10240
BW = NP // NW
DEAD = BW
CHUNK = 128
EPAD = 327680
SCAN = 2048
CCAP = 2304
ECAP = EPAD + CHUNK
NBLK = 512

@functools.cache
def _mesh():
    return plsc.VectorSubcoreMesh(
        core_axis_name="c", subcore_axis_name="s",
        num_cores=NC, num_subcores=NS)

_f32 = jnp.float32
_i32 = jnp.int32


def _seg_sum_body(split_edges, W, tbl, srcs, dst, zero, out,
                  sidx, didx, rows, acc, sem):
    c = lax.axis_index("c")
    s = lax.axis_index("s")
    rows_per_sub = NP // NS
    r0 = s * rows_per_sub
    pltpu.sync_copy(zero.at[pl.ds(r0, rows_per_sub)],
                    acc.at[pl.ds(r0, rows_per_sub)])
    plsc.subcore_barrier()

    if split_edges:
        wid = c * NS + s
        nchunks = EPAD // NW // CHUNK
        ebase = wid * (EPAD // NW)
        soff = 0
    else:
        nchunks = EPAD // NS // CHUNK
        ebase = s * (EPAD // NS)
        soff = c * EPAD

    def chunk_body(i, carry):
        b = ebase + i * CHUNK
        pltpu.sync_copy(srcs.at[pl.ds(soff + b, CHUNK)], sidx)
        pltpu.sync_copy(dst.at[pl.ds(b, CHUNK)], didx)
        pltpu.async_copy(tbl.at[sidx], rows, sem).wait()
        pltpu.sync_copy(rows, acc.at[didx], add=True)
        return carry

    lax.fori_loop(0, nchunks, chunk_body, 0)
    plsc.subcore_barrier()
    pltpu.sync_copy(acc.at[pl.ds(r0, rows_per_sub)],
                    out.at[c, pl.ds(r0, rows_per_sub)])


def _seg_sum_sc(tbl, srcs, dst, split_edges):
    W = tbl.shape[-1]
    zero = jnp.zeros((NP, W), _f32)
    body = functools.partial(_seg_sum_body, split_edges, W)
    return pl.kernel(
        body,
        out_type=jax.ShapeDtypeStruct((NC, NP, W), _f32),
        mesh=_mesh(),
        scratch_types=[
            pltpu.VMEM((CHUNK,), _i32),
            pltpu.VMEM((CHUNK,), _i32),
            pltpu.VMEM((CHUNK, W), _f32),
            pltpu.VMEM_SHARED((NP, W), _f32),
            pltpu.SemaphoreType.DMA,
        ],
    )(tbl, srcs, dst, zero)


SLP = 640
EPW = EPAD // NW
CAPR = 16 + EPW
PKN = NW * NW * CAPR
DEADPK = DEAD << 14
TBLR = 16384


def _bucket_body(srcs, dst, pk, sbuf, dbuf, vpage, cvec, fills, offs):
    c = lax.axis_index("c")
    s = lax.axis_index("s")
    wid = c * NS + s
    iota = lax.iota(_i32, 16)
    for b in range(NW):
        fills[b] = 0
        offs[b] = 0
    eb = wid * EPW

    def chunk_body(ci, _):
        pltpu.sync_copy(srcs.at[pl.ds(eb + ci * SCAN, SCAN)], sbuf)
        pltpu.sync_copy(dst.at[pl.ds(eb + ci * SCAN, SCAN)], dbuf)

        def vec_body(j, _):
            dv = dbuf[pl.ds(j * 16, 16)]
            sv = sbuf[pl.ds(j * 16, 16)]
            bv = (dv * 13108) >> 22
            pkv = sv | ((dv - bv * BW) << 14)
            for i in range(16):
                b = bv[i]
                x = pkv[i]
                f = fills[b]
                base = pl.multiple_of(b * SLP + ((f >> 4) << 4), 16)
                cur = vpage[pl.ds(base, 16)]
                vpage[pl.ds(base, 16)] = jnp.where(iota == (f & 15), x, cur)
                fills[b] = f + 1

                @pl.when(f + 1 == SLP)
                def _():
                    o = offs[b]
                    doff = pl.multiple_of(
                        (b * NW + wid) * CAPR + 16 + o * SLP, 16)
                    pltpu.sync_copy(
                        vpage.at[pl.ds(pl.multiple_of(b * SLP, 16), SLP)],
                        pk.at[pl.ds(doff, SLP)])
                    offs[b] = o + 1
                    fills[b] = 0

            return 0

        lax.fori_loop(0, SCAN // 16, vec_body, 0)
        return 0

    lax.fori_loop(0, EPW // SCAN, chunk_body, 0)
    for b in range(NW):
        f = fills[b]
        o = offs[b]

        @pl.when(f > 0)
        def _():
            doff = pl.multiple_of((b * NW + wid) * CAPR + 16 + o * SLP, 16)
            pltpu.sync_copy(vpage.at[pl.ds(b * SLP, SLP)],
                            pk.at[pl.ds(doff, SLP)])

        cvec[pl.ds(0, 16)] = jnp.zeros((16,), _i32) + (o * SLP + f)
        pltpu.sync_copy(cvec, pk.at[pl.ds((b * NW + wid) * CAPR, 16)])


def _bucket_sc(srcs, dst):
    return pl.kernel(
        _bucket_body,
        out_type=jax.ShapeDtypeStruct((PKN,), _i32),
        mesh=_mesh(),
        scratch_types=[
            pltpu.VMEM((SCAN,), _i32),
            pltpu.VMEM((SCAN,), _i32),
            pltpu.VMEM((NW * SLP,), _i32),
            pltpu.VMEM((16,), _i32),
            pltpu.SMEM((NW,), _i32),
            pltpu.SMEM((NW,), _i32),
        ],
    )(srcs, dst)


def _max_block(tbl, pk, ebase, cnt, k, pbuf, sidx, didx, rows, acc, sem,
               posloc=None, W=128):
    iota = lax.iota(_i32, 16)
    off = pl.multiple_of(ebase + k * CHUNK, 16)
    pltpu.sync_copy(pk.at[pl.ds(off, CHUNK)], pbuf)
    for m in range(CHUNK // 16):
        pv = pbuf[pl.ds(m * 16, 16)]
        pv = jnp.where(k * CHUNK + m * 16 + iota >= cnt, DEADPK, pv)
        sidx[pl.ds(m * 16, 16)] = pv & 16383
        didx[pl.ds(m * 16, 16)] = pv >> 14
    pltpu.async_copy(tbl.at[sidx], rows, sem).wait()

    def sub_body(j, _):
        dv = didx[pl.ds(j * 16, 16)]
        for i in range(16):
            d = dv[i]
            if posloc is None:
                for g in range(W // 16):
                    sl = pl.ds(g * 16, 16)
                    acc[d, sl] = jnp.maximum(acc[d, sl], rows[j * 16 + i, sl])
            else:
                sl = pl.ds(0, 16)
                val = rows[j * 16 + i, sl] - posloc[d, sl]
                acc[d, sl] = jnp.maximum(acc[d, sl], val)
        return 0

    lax.fori_loop(0, CHUNK // 16, sub_body, 0)


def _seg_max_body(W, tbl, pk, out, pbuf, sidx, didx, rows, acc, sem):
    c = lax.axis_index("c")
    s = lax.axis_index("s")
    wid = c * NS + s
    ninf = jnp.full((16,), -jnp.inf, _f32)

    def init_body(r, _):
        for g in range(W // 16):
            acc[r, pl.ds(g * 16, 16)] = ninf
        return 0

    lax.fori_loop(0, BW + 1, init_body, 0)

    def per_writer(w, _):
        rbase = pl.multiple_of((wid * NW + w) * CAPR, 16)
        pltpu.sync_copy(pk.at[pl.ds(rbase, 16)], pbuf.at[pl.ds(0, 16)])
        cnt = pbuf[pl.ds(0, 16)][0]
        nblk = (cnt + CHUNK - 1) // CHUNK

        def blk_body(k, _):
            _max_block(tbl, pk, rbase + 16, cnt, k, pbuf, sidx, didx, rows,
                       acc, sem, W=W)
            return 0

        lax.fori_loop(0, nblk, blk_body, 0)
        return 0

    lax.fori_loop(0, NW, per_writer, 0)
    pltpu.sync_copy(acc.at[pl.ds(0, BW)],
                    out.at[pl.ds(pl.multiple_of(wid * BW, 8), BW)])


def _seg_max_sc(tbl, pk):
    W = tbl.shape[-1]
    body = functools.partial(_seg_max_body, W)
    return pl.kernel(
        body,
        out_type=jax.ShapeDtypeStruct((NP, W), _f32),
        mesh=_mesh(),
        scratch_types=[
            pltpu.VMEM((CHUNK,), _i32),
            pltpu.VMEM((CHUNK,), _i32),
            pltpu.VMEM((CHUNK,), _i32),
            pltpu.VMEM((CHUNK, W), _f32),
            pltpu.VMEM((BW + 1, W), _f32),
            pltpu.SemaphoreType.DMA,
        ],
    )(tbl, pk)


def _rel_max_body(pos128, pk, out, pbuf, sidx, didx, rows, posloc, acc, sem):
    c = lax.axis_index("c")
    s = lax.axis_index("s")
    wid = c * NS + s
    lo = wid * BW
    ninf = jnp.full((16,), -jnp.inf, _f32)

    def init_body(r, _):
        acc[r, pl.ds(0, 16)] = ninf
        return 0

    lax.fori_loop(0, BW + 1, init_body, 0)
    pltpu.sync_copy(pos128.at[pl.ds(pl.multiple_of(lo, 8), BW)],
                    posloc.at[pl.ds(0, BW)])

    def per_writer(w, _):
        rbase = pl.multiple_of((wid * NW + w) * CAPR, 16)
        pltpu.sync_copy(pk.at[pl.ds(rbase, 16)], pbuf.at[pl.ds(0, 16)])
        cnt = pbuf[pl.ds(0, 16)][0]
        nblk = (cnt + CHUNK - 1) // CHUNK

        def blk_body(k, _):
            _max_block(pos128, pk, rbase + 16, cnt, k, pbuf, sidx, didx, rows,
                       acc, sem, posloc=posloc, W=16)
            return 0

        lax.fori_loop(0, nblk, blk_body, 0)
        return 0

    lax.fori_loop(0, NW, per_writer, 0)
    pltpu.sync_copy(acc.at[pl.ds(0, BW)],
                    out.at[pl.ds(pl.multiple_of(wid * BW, 8), BW)])


def _rel_max_sc(pos128, pk):
    return pl.kernel(
        _rel_max_body,
        out_type=jax.ShapeDtypeStruct((NP, 16), _f32),
        mesh=_mesh(),
        scratch_types=[
            pltpu.VMEM((CHUNK,), _i32),
            pltpu.VMEM((CHUNK,), _i32),
            pltpu.VMEM((CHUNK,), _i32),
            pltpu.VMEM((CHUNK, 128), _f32),
            pltpu.VMEM((BW + 1, 128), _f32),
            pltpu.VMEM((BW + 1, 16), _f32),
            pltpu.SemaphoreType.DMA,
        ],
    )(pos128, pk)


def _dot(a, b):
    return jnp.dot(a, b, preferred_element_type=_f32)


def _gin1_body(x_ref, p_ref, w1_ref, b1_ref, wn_ref, bn_ref, h_ref, sch_ref):
    xb = x_ref[...]
    xin = xb + p_ref[0] + p_ref[1]
    h = jnp.maximum(_dot(xin, w1_ref[...]) + b1_ref[...], 0.0)
    h_ref[0] = h[:, :128]
    h_ref[1] = h[:, 128:]
    sch_ref[...] = _dot(xb, wn_ref[...]) + bn_ref[...]


def _tc_gin1(xp, part, W1, b1, Wn, bn):
    grid = (NP // NBLK,)
    return pl.pallas_call(
        _gin1_body,
        grid=grid,
        in_specs=[
            pl.BlockSpec((NBLK, 128), lambda i: (i, 0)),
            pl.BlockSpec((2, NBLK, 128), lambda i: (0, i, 0)),
            pl.BlockSpec((128, 256), lambda i: (0, 0)),
            pl.BlockSpec((1, 256), lambda i: (0, 0)),
            pl.BlockSpec((128, 128), lambda i: (0, 0)),
            pl.BlockSpec((1, 128), lambda i: (0, 0)),
        ],
        out_specs=[
            pl.BlockSpec((2, NBLK, 128), lambda i: (0, i, 0)),
            pl.BlockSpec((NBLK, 128), lambda i: (i, 0)),
        ],
        out_shape=[
            jax.ShapeDtypeStruct((2, NP, 128), _f32),
            jax.ShapeDtypeStruct((NP, 128), _f32),
        ],
    )(xp, part, W1, b1.reshape(1, -1), Wn, bn.reshape(1, -1))


def _gin2_body(h_ref, a_ref, w2_ref, b2_ref, bat_ref, gnn_ref):
    h1 = jnp.concatenate([h_ref[0], h_ref[1]], axis=1)
    a2 = jnp.concatenate([a_ref[0], a_ref[1]], axis=1)
    h2 = jnp.maximum(_dot(h1 + a2, w2_ref[...]) + b2_ref[...], 0.0)
    bv = bat_ref[...][0, 0]
    oh = (lax.broadcasted_iota(_i32, (G, NBLK), 0) == bv[None, :]).astype(_f32)

    @pl.when(pl.program_id(0) == 0)
    def _():
        gnn_ref[...] = jnp.zeros_like(gnn_ref)

    gnn_ref[...] += _dot(oh, h2)


def _tc_gin2pool(hcat, half, W2, b2, bat3):
    grid = (NP // NBLK,)
    return pl.pallas_call(
        _gin2_body,
        grid=grid,
        in_specs=[
            pl.BlockSpec((2, NBLK, 128), lambda i: (0, i, 0)),
            pl.BlockSpec((2, NBLK, 128), lambda i: (0, i, 0)),
            pl.BlockSpec((256, 256), lambda i: (0, 0)),
            pl.BlockSpec((1, 256), lambda i: (0, 0)),
            pl.BlockSpec((1, 1, NBLK), lambda i: (i, 0, 0)),
        ],
        out_specs=pl.BlockSpec((G, 256), lambda i: (0, 0)),
        out_shape=jax.ShapeDtypeStruct((G, 256), _f32),
    )(hcat, half, W2, b2.reshape(1, -1), bat3)


def _spmm_body(pool, m_ref, r_ref, w1_ref, w2_ref, b_ref, bat_ref, o_ref):
    m = m_ref[...]
    r = r_ref[...]
    m = jnp.where(m == -jnp.inf, 0.0, m)
    r = jnp.where(r == -jnp.inf, 0.0, r)
    sch = _dot(m, w1_ref[...]) + _dot(r, w2_ref[...]) + b_ref[...]
    if pool:
        bv = bat_ref[...][0, 0]
        oh = (lax.broadcasted_iota(_i32, (G, NBLK), 0)
              == bv[None, :]).astype(_f32)

        @pl.when(pl.program_id(0) == 0)
        def _():
            o_ref[...] = jnp.zeros_like(o_ref)

        o_ref[...] += _dot(oh, sch)
    else:
        o_ref[...] = sch


def _tc_spmm(maxsch, relmax, W1, W2, b, bat3, pool):
    grid = (NP // NBLK,)
    in_specs = [
        pl.BlockSpec((NBLK, 128), lambda i: (i, 0)),
        pl.BlockSpec((NBLK, 16), lambda i: (i, 0)),
        pl.BlockSpec((128, 128), lambda i: (0, 0)),
        pl.BlockSpec((16, 128), lambda i: (0, 0)),
        pl.BlockSpec((1, 128), lambda i: (0, 0)),
        pl.BlockSpec((1, 1, NBLK), lambda i: (i, 0, 0)),
    ]
    if pool:
        out_specs = pl.BlockSpec((G, 128), lambda i: (0, 0))
        out_shape = jax.ShapeDtypeStruct((G, 128), _f32)
    else:
        out_specs = pl.BlockSpec((NBLK, 128), lambda i: (i, 0))
        out_shape = jax.ShapeDtypeStruct((NP, 128), _f32)
    return pl.pallas_call(
        functools.partial(_spmm_body, pool),
        grid=grid,
        in_specs=in_specs,
        out_specs=out_specs,
        out_shape=out_shape,
    )(maxsch, relmax, W1, W2, b.reshape(1, -1), bat3)


def _tail_body(gnn_ref, sp_ref, fp_ref, wg_ref, bg_ref, wf1_ref, bf1_ref,
               wf2_ref, bf2_ref, wfl_ref, bfl_ref, wh_ref, bh_ref, wo_ref,
               bo_ref, out_ref, gnnout_ref):
    gnn_out = _dot(gnn_ref[...], wg_ref[...]) + bg_ref[...]
    fcn = _dot(jnp.maximum(_dot(fp_ref[...], wf1_ref[...]) + bf1_ref[...],
                           0.0), wf2_ref[...]) + bf2_ref[...]
    fcn_out = _dot(fcn, wfl_ref[...]) + bfl_ref[...]
    cat = jnp.concatenate([gnn_out, fcn_out, sp_ref[...]], axis=1)
    hid = jnp.maximum(_dot(cat, wh_ref[...]) + bh_ref[...], 0.0)
    out_ref[...] = _dot(hid, wo_ref[...]) + bo_ref[...]
    gnnout_ref[...] = gnn_out


def _tc_tail(gnn, schpool, fp, Wg, bg, Wf1, bf1, Wf2, bf2, Wfl, bfl, Wh, bh,
             Wo128, bo128):
    return pl.pallas_call(
        _tail_body,
        out_shape=[
            jax.ShapeDtypeStruct((G, 128), _f32),
            jax.ShapeDtypeStruct((G, 256), _f32),
        ],
    )(gnn, schpool, fp, Wg, bg.reshape(1, -1), Wf1, bf1.reshape(1, -1),
      Wf2, bf2.reshape(1, -1), Wfl, bfl.reshape(1, -1), Wh, bh.reshape(1, -1),
      Wo128, bo128)


def kernel(x, pos, fp, edge_index, batch,
           W_gin1, b_gin1, W_gin2, b_gin2, W_gcn, b_gcn,
           W_f1, b_f1, W_f2, b_f2, W_fl, b_fl,
           W_n2h, b_n2h, W_sp, b_sp, W_hid, b_hid, W_out, b_out):
    src = edge_index[0].astype(_i32)
    dst = edge_index[1].astype(_i32)
    npad = EPAD - E
    srcp = jnp.concatenate([src, jnp.zeros((npad,), _i32)])
    dstp = jnp.concatenate([dst, jnp.full((npad,), NP - 1, _i32)])
    srcs2 = jnp.concatenate([srcp, srcp + NP])

    xp = jnp.concatenate([x, jnp.zeros((NP - N, 128), _f32)])
    pos128 = jnp.zeros((TBLR, 128), _f32).at[:N, :3].set(pos)
    batp3 = jnp.concatenate([batch.astype(_i32),
                             jnp.full((NP - N,), G, _i32)]).reshape(-1, 1, NBLK)

    pk = _bucket_sc(srcp, dstp)

    part = _seg_sum_sc(xp, srcp, dstp, split_edges=True)
    hcat, sch = _tc_gin1(xp, part, W_gin1, b_gin1, W_n2h, b_n2h)

    half = _seg_sum_sc(hcat.reshape(2 * NP, 128), srcs2, dstp,
                       split_edges=False)
    gnn = _tc_gin2pool(hcat, half, W_gin2, b_gin2, batp3)

    relmax = _rel_max_sc(pos128, pk)
    Wsp1 = W_sp[:128]
    Wsp2 = jnp.zeros((16, 128), _f32).at[:3].set(W_sp[128:])
    for i in range(3):
        sch16 = jnp.concatenate([sch, jnp.zeros((TBLR - NP, 128), _f32)])
        maxsch = _seg_max_sc(sch16, pk)
        schout = _tc_spmm(maxsch, relmax, Wsp1, Wsp2, b_sp, batp3,
                          pool=(i == 2))
        if i < 2:
            sch = schout
    sch_out = schout

    Wo128 = jnp.zeros((128, 128), _f32).at[:, :2].set(W_out)
    bo128 = jnp.zeros((1, 128), _f32).at[0, :2].set(b_out)
    out128, gnn_out = _tc_tail(gnn, sch_out, fp, W_gcn, b_gcn, W_f1, b_f1,
                               W_f2, b_f2, W_fl, b_fl, W_hid, b_hid,
                               Wo128, bo128)
    return (out128[:, :2], gnn_out, sch_out)


# speedup vs baseline: 1.0175x; 1.0175x over previous
import functools

import jax
import jax.numpy as jnp
from jax import lax
from jax.experimental import pallas as pl
from jax.experimental.pallas import tpu as pltpu
from jax.experimental.pallas import tpu_sc as plsc

N = 10000
E = 320000
G = 64

NC = 2
NS = 16
NW = NC * NS

NP = 10240
BW = NP // NW
DEAD = BW
CHUNK = 128
EPAD = 327680
SCAN = 2048
CCAP = 2304
ECAP = EPAD + CHUNK
NBLK = 512

@functools.cache
def _mesh():
    return plsc.VectorSubcoreMesh(
        core_axis_name="c", subcore_axis_name="s",
        num_cores=NC, num_subcores=NS)

_f32 = jnp.float32
_i32 = jnp.int32


def _seg_sum_body(split_edges, W, tbl, srcs, dst, zero, out,
                  sidx0, sidx1, didx0, didx1, rows0, rows1, acc,
                  gs0, gs1, ss0, ss1):
    c = lax.axis_index("c")
    s = lax.axis_index("s")
    rows_per_sub = NP // NS
    r0 = s * rows_per_sub
    pltpu.sync_copy(zero.at[pl.ds(r0, rows_per_sub)],
                    acc.at[pl.ds(r0, rows_per_sub)])
    plsc.subcore_barrier()

    if split_edges:
        wid = c * NS + s
        nchunks = EPAD // NW // CHUNK
        ebase = wid * (EPAD // NW)
        soff = 0
    else:
        nchunks = EPAD // NS // CHUNK
        ebase = s * (EPAD // NS)
        soff = c * EPAD

    def pair_body(p, carry):
        b0 = ebase + (2 * p) * CHUNK
        b1 = b0 + CHUNK
        pltpu.sync_copy(srcs.at[pl.ds(soff + b0, CHUNK)], sidx0)
        pltpu.sync_copy(dst.at[pl.ds(b0, CHUNK)], didx0)
        g0 = pltpu.async_copy(tbl.at[sidx0], rows0, gs0)
        pltpu.sync_copy(srcs.at[pl.ds(soff + b1, CHUNK)], sidx1)
        pltpu.sync_copy(dst.at[pl.ds(b1, CHUNK)], didx1)
        g1 = pltpu.async_copy(tbl.at[sidx1], rows1, gs1)
        g0.wait()
        s0 = pltpu.async_copy(rows0, acc.at[didx0], ss0, add=True)
        g1.wait()
        s1 = pltpu.async_copy(rows1, acc.at[didx1], ss1, add=True)
        s0.wait()
        s1.wait()
        return carry

    lax.fori_loop(0, nchunks // 2, pair_body, 0)
    plsc.subcore_barrier()
    pltpu.sync_copy(acc.at[pl.ds(r0, rows_per_sub)],
                    out.at[c, pl.ds(r0, rows_per_sub)])


def _seg_sum_sc(tbl, srcs, dst, split_edges):
    W = tbl.shape[-1]
    zero = jnp.zeros((NP, W), _f32)
    body = functools.partial(_seg_sum_body, split_edges, W)
    return pl.kernel(
        body,
        out_type=jax.ShapeDtypeStruct((NC, NP, W), _f32),
        mesh=_mesh(),
        scratch_types=[
            pltpu.VMEM((CHUNK,), _i32),
            pltpu.VMEM((CHUNK,), _i32),
            pltpu.VMEM((CHUNK,), _i32),
            pltpu.VMEM((CHUNK,), _i32),
            pltpu.VMEM((CHUNK, W), _f32),
            pltpu.VMEM((CHUNK, W), _f32),
            pltpu.VMEM_SHARED((NP, W), _f32),
            pltpu.SemaphoreType.DMA,
            pltpu.SemaphoreType.DMA,
            pltpu.SemaphoreType.DMA,
            pltpu.SemaphoreType.DMA,
        ],
    )(tbl, srcs, dst, zero)


SLP = 640
EPW = EPAD // NW
CAPR = 16 + EPW
PKN = NW * NW * CAPR
DEADPK = DEAD << 14
TBLR = 16384


def _bucket_body(srcs, dst, pk, sbuf, dbuf, vpage, cvec, fills, offs):
    c = lax.axis_index("c")
    s = lax.axis_index("s")
    wid = c * NS + s
    iota = lax.iota(_i32, 16)
    for b in range(NW):
        fills[b] = 0
        offs[b] = 0
    eb = wid * EPW

    def chunk_body(ci, _):
        pltpu.sync_copy(srcs.at[pl.ds(eb + ci * SCAN, SCAN)], sbuf)
        pltpu.sync_copy(dst.at[pl.ds(eb + ci * SCAN, SCAN)], dbuf)

        def vec_body(j, _):
            dv = dbuf[pl.ds(j * 16, 16)]
            sv = sbuf[pl.ds(j * 16, 16)]
            bv = (dv * 13108) >> 22
            pkv = sv | ((dv - bv * BW) << 14)
            for i in range(16):
                b = bv[i]
                x = pkv[i]
                f = fills[b]
                base = pl.multiple_of(b * SLP + ((f >> 4) << 4), 16)
                cur = vpage[pl.ds(base, 16)]
                vpage[pl.ds(base, 16)] = jnp.where(iota == (f & 15), x, cur)
                fills[b] = f + 1

                @pl.when(f + 1 == SLP)
                def _():
                    o = offs[b]
                    doff = pl.multiple_of(
                        (b * NW + wid) * CAPR + 16 + o * SLP, 16)
                    pltpu.sync_copy(
                        vpage.at[pl.ds(pl.multiple_of(b * SLP, 16), SLP)],
                        pk.at[pl.ds(doff, SLP)])
                    offs[b] = o + 1
                    fills[b] = 0

            return 0

        lax.fori_loop(0, SCAN // 16, vec_body, 0)
        return 0

    lax.fori_loop(0, EPW // SCAN, chunk_body, 0)
    for b in range(NW):
        f = fills[b]
        o = offs[b]

        @pl.when(f > 0)
        def _():
            doff = pl.multiple_of((b * NW + wid) * CAPR + 16 + o * SLP, 16)
            pltpu.sync_copy(vpage.at[pl.ds(b * SLP, SLP)],
                            pk.at[pl.ds(doff, SLP)])

        cvec[pl.ds(0, 16)] = jnp.zeros((16,), _i32) + (o * SLP + f)
        pltpu.sync_copy(cvec, pk.at[pl.ds((b * NW + wid) * CAPR, 16)])


def _bucket_sc(srcs, dst):
    return pl.kernel(
        _bucket_body,
        out_type=jax.ShapeDtypeStruct((PKN,), _i32),
        mesh=_mesh(),
        scratch_types=[
            pltpu.VMEM((SCAN,), _i32),
            pltpu.VMEM((SCAN,), _i32),
            pltpu.VMEM((NW * SLP,), _i32),
            pltpu.VMEM((16,), _i32),
            pltpu.SMEM((NW,), _i32),
            pltpu.SMEM((NW,), _i32),
        ],
    )(srcs, dst)


def _max_block(tbl, pk, ebase, cnt, k, pbuf, sidx, didx, rows, acc, sem,
               posloc=None, W=128):
    iota = lax.iota(_i32, 16)
    off = pl.multiple_of(ebase + k * CHUNK, 16)
    pltpu.sync_copy(pk.at[pl.ds(off, CHUNK)], pbuf)
    for m in range(CHUNK // 16):
        pv = pbuf[pl.ds(m * 16, 16)]
        pv = jnp.where(k * CHUNK + m * 16 + iota >= cnt, DEADPK, pv)
        sidx[pl.ds(m * 16, 16)] = pv & 16383
        didx[pl.ds(m * 16, 16)] = pv >> 14
    pltpu.async_copy(tbl.at[sidx], rows, sem).wait()

    def sub_body(j, _):
        dv = didx[pl.ds(j * 16, 16)]
        for i in range(16):
            d = dv[i]
            if posloc is None:
                for g in range(W // 16):
                    sl = pl.ds(g * 16, 16)
                    acc[d, sl] = jnp.maximum(acc[d, sl], rows[j * 16 + i, sl])
            else:
                sl = pl.ds(0, 16)
                val = rows[j * 16 + i, sl] - posloc[d, sl]
                acc[d, sl] = jnp.maximum(acc[d, sl], val)
        return 0

    lax.fori_loop(0, CHUNK // 16, sub_body, 0)


def _seg_max_body(W, tbl, pk, out, pbuf, sidx, didx, rows, acc, sem):
    c = lax.axis_index("c")
    s = lax.axis_index("s")
    wid = c * NS + s
    ninf = jnp.full((16,), -jnp.inf, _f32)

    def init_body(r, _):
        for g in range(W // 16):
            acc[r, pl.ds(g * 16, 16)] = ninf
        return 0

    lax.fori_loop(0, BW + 1, init_body, 0)

    def per_writer(w, _):
        rbase = pl.multiple_of((wid * NW + w) * CAPR, 16)
        pltpu.sync_copy(pk.at[pl.ds(rbase, 16)], pbuf.at[pl.ds(0, 16)])
        cnt = pbuf[pl.ds(0, 16)][0]
        nblk = (cnt + CHUNK - 1) // CHUNK

        def blk_body(k, _):
            _max_block(tbl, pk, rbase + 16, cnt, k, pbuf, sidx, didx, rows,
                       acc, sem, W=W)
            return 0

        lax.fori_loop(0, nblk, blk_body, 0)
        return 0

    lax.fori_loop(0, NW, per_writer, 0)
    pltpu.sync_copy(acc.at[pl.ds(0, BW)],
                    out.at[pl.ds(pl.multiple_of(wid * BW, 8), BW)])


def _seg_max_sc(tbl, pk):
    W = tbl.shape[-1]
    body = functools.partial(_seg_max_body, W)
    return pl.kernel(
        body,
        out_type=jax.ShapeDtypeStruct((NP, W), _f32),
        mesh=_mesh(),
        scratch_types=[
            pltpu.VMEM((CHUNK,), _i32),
            pltpu.VMEM((CHUNK,), _i32),
            pltpu.VMEM((CHUNK,), _i32),
            pltpu.VMEM((CHUNK, W), _f32),
            pltpu.VMEM((BW + 1, W), _f32),
            pltpu.SemaphoreType.DMA,
        ],
    )(tbl, pk)


def _rel_max_body(pos128, pk, out, pbuf, sidx, didx, rows, posloc, acc, sem):
    c = lax.axis_index("c")
    s = lax.axis_index("s")
    wid = c * NS + s
    lo = wid * BW
    ninf = jnp.full((16,), -jnp.inf, _f32)

    def init_body(r, _):
        acc[r, pl.ds(0, 16)] = ninf
        return 0

    lax.fori_loop(0, BW + 1, init_body, 0)
    pltpu.sync_copy(pos128.at[pl.ds(pl.multiple_of(lo, 8), BW)],
                    posloc.at[pl.ds(0, BW)])

    def per_writer(w, _):
        rbase = pl.multiple_of((wid * NW + w) * CAPR, 16)
        pltpu.sync_copy(pk.at[pl.ds(rbase, 16)], pbuf.at[pl.ds(0, 16)])
        cnt = pbuf[pl.ds(0, 16)][0]
        nblk = (cnt + CHUNK - 1) // CHUNK

        def blk_body(k, _):
            _max_block(pos128, pk, rbase + 16, cnt, k, pbuf, sidx, didx, rows,
                       acc, sem, posloc=posloc, W=16)
            return 0

        lax.fori_loop(0, nblk, blk_body, 0)
        return 0

    lax.fori_loop(0, NW, per_writer, 0)
    pltpu.sync_copy(acc.at[pl.ds(0, BW)],
                    out.at[pl.ds(pl.multiple_of(wid * BW, 8), BW)])


def _rel_max_sc(pos128, pk):
    return pl.kernel(
        _rel_max_body,
        out_type=jax.ShapeDtypeStruct((NP, 16), _f32),
        mesh=_mesh(),
        scratch_types=[
            pltpu.VMEM((CHUNK,), _i32),
            pltpu.VMEM((CHUNK,), _i32),
            pltpu.VMEM((CHUNK,), _i32),
            pltpu.VMEM((CHUNK, 128), _f32),
            pltpu.VMEM((BW + 1, 128), _f32),
            pltpu.VMEM((BW + 1, 16), _f32),
            pltpu.SemaphoreType.DMA,
        ],
    )(pos128, pk)


def _dot(a, b):
    return jnp.dot(a, b, preferred_element_type=_f32)


def _gin1_body(x_ref, p_ref, w1_ref, b1_ref, wn_ref, bn_ref, h_ref, sch_ref):
    xb = x_ref[...]
    xin = xb + p_ref[0] + p_ref[1]
    h = jnp.maximum(_dot(xin, w1_ref[...]) + b1_ref[...], 0.0)
    h_ref[0] = h[:, :128]
    h_ref[1] = h[:, 128:]
    sch_ref[...] = _dot(xb, wn_ref[...]) + bn_ref[...]


def _tc_gin1(xp, part, W1, b1, Wn, bn):
    grid = (NP // NBLK,)
    return pl.pallas_call(
        _gin1_body,
        grid=grid,
        in_specs=[
            pl.BlockSpec((NBLK, 128), lambda i: (i, 0)),
            pl.BlockSpec((2, NBLK, 128), lambda i: (0, i, 0)),
            pl.BlockSpec((128, 256), lambda i: (0, 0)),
            pl.BlockSpec((1, 256), lambda i: (0, 0)),
            pl.BlockSpec((128, 128), lambda i: (0, 0)),
            pl.BlockSpec((1, 128), lambda i: (0, 0)),
        ],
        out_specs=[
            pl.BlockSpec((2, NBLK, 128), lambda i: (0, i, 0)),
            pl.BlockSpec((NBLK, 128), lambda i: (i, 0)),
        ],
        out_shape=[
            jax.ShapeDtypeStruct((2, NP, 128), _f32),
            jax.ShapeDtypeStruct((NP, 128), _f32),
        ],
    )(xp, part, W1, b1.reshape(1, -1), Wn, bn.reshape(1, -1))


def _gin2_body(h_ref, a_ref, w2_ref, b2_ref, bat_ref, gnn_ref):
    h1 = jnp.concatenate([h_ref[0], h_ref[1]], axis=1)
    a2 = jnp.concatenate([a_ref[0], a_ref[1]], axis=1)
    h2 = jnp.maximum(_dot(h1 + a2, w2_ref[...]) + b2_ref[...], 0.0)
    bv = bat_ref[...][0, 0]
    oh = (lax.broadcasted_iota(_i32, (G, NBLK), 0) == bv[None, :]).astype(_f32)

    @pl.when(pl.program_id(0) == 0)
    def _():
        gnn_ref[...] = jnp.zeros_like(gnn_ref)

    gnn_ref[...] += _dot(oh, h2)


def _tc_gin2pool(hcat, half, W2, b2, bat3):
    grid = (NP // NBLK,)
    return pl.pallas_call(
        _gin2_body,
        grid=grid,
        in_specs=[
            pl.BlockSpec((2, NBLK, 128), lambda i: (0, i, 0)),
            pl.BlockSpec((2, NBLK, 128), lambda i: (0, i, 0)),
            pl.BlockSpec((256, 256), lambda i: (0, 0)),
            pl.BlockSpec((1, 256), lambda i: (0, 0)),
            pl.BlockSpec((1, 1, NBLK), lambda i: (i, 0, 0)),
        ],
        out_specs=pl.BlockSpec((G, 256), lambda i: (0, 0)),
        out_shape=jax.ShapeDtypeStruct((G, 256), _f32),
    )(hcat, half, W2, b2.reshape(1, -1), bat3)


def _spmm_body(pool, m_ref, r_ref, w1_ref, w2_ref, b_ref, bat_ref, o_ref):
    m = m_ref[...]
    r = r_ref[...]
    m = jnp.where(m == -jnp.inf, 0.0, m)
    r = jnp.where(r == -jnp.inf, 0.0, r)
    sch = _dot(m, w1_ref[...]) + _dot(r, w2_ref[...]) + b_ref[...]
    if pool:
        bv = bat_ref[...][0, 0]
        oh = (lax.broadcasted_iota(_i32, (G, NBLK), 0)
              == bv[None, :]).astype(_f32)

        @pl.when(pl.program_id(0) == 0)
        def _():
            o_ref[...] = jnp.zeros_like(o_ref)

        o_ref[...] += _dot(oh, sch)
    else:
        o_ref[...] = sch


def _tc_spmm(maxsch, relmax, W1, W2, b, bat3, pool):
    grid = (NP // NBLK,)
    in_specs = [
        pl.BlockSpec((NBLK, 128), lambda i: (i, 0)),
        pl.BlockSpec((NBLK, 16), lambda i: (i, 0)),
        pl.BlockSpec((128, 128), lambda i: (0, 0)),
        pl.BlockSpec((16, 128), lambda i: (0, 0)),
        pl.BlockSpec((1, 128), lambda i: (0, 0)),
        pl.BlockSpec((1, 1, NBLK), lambda i: (i, 0, 0)),
    ]
    if pool:
        out_specs = pl.BlockSpec((G, 128), lambda i: (0, 0))
        out_shape = jax.ShapeDtypeStruct((G, 128), _f32)
    else:
        out_specs = pl.BlockSpec((NBLK, 128), lambda i: (i, 0))
        out_shape = jax.ShapeDtypeStruct((NP, 128), _f32)
    return pl.pallas_call(
        functools.partial(_spmm_body, pool),
        grid=grid,
        in_specs=in_specs,
        out_specs=out_specs,
        out_shape=out_shape,
    )(maxsch, relmax, W1, W2, b.reshape(1, -1), bat3)


def _tail_body(gnn_ref, sp_ref, fp_ref, wg_ref, bg_ref, wf1_ref, bf1_ref,
               wf2_ref, bf2_ref, wfl_ref, bfl_ref, wh_ref, bh_ref, wo_ref,
               bo_ref, out_ref, gnnout_ref):
    gnn_out = _dot(gnn_ref[...], wg_ref[...]) + bg_ref[...]
    fcn = _dot(jnp.maximum(_dot(fp_ref[...], wf1_ref[...]) + bf1_ref[...],
                           0.0), wf2_ref[...]) + bf2_ref[...]
    fcn_out = _dot(fcn, wfl_ref[...]) + bfl_ref[...]
    cat = jnp.concatenate([gnn_out, fcn_out, sp_ref[...]], axis=1)
    hid = jnp.maximum(_dot(cat, wh_ref[...]) + bh_ref[...], 0.0)
    out_ref[...] = _dot(hid, wo_ref[...]) + bo_ref[...]
    gnnout_ref[...] = gnn_out


def _tc_tail(gnn, schpool, fp, Wg, bg, Wf1, bf1, Wf2, bf2, Wfl, bfl, Wh, bh,
             Wo128, bo128):
    return pl.pallas_call(
        _tail_body,
        out_shape=[
            jax.ShapeDtypeStruct((G, 128), _f32),
            jax.ShapeDtypeStruct((G, 256), _f32),
        ],
    )(gnn, schpool, fp, Wg, bg.reshape(1, -1), Wf1, bf1.reshape(1, -1),
      Wf2, bf2.reshape(1, -1), Wfl, bfl.reshape(1, -1), Wh, bh.reshape(1, -1),
      Wo128, bo128)


def kernel(x, pos, fp, edge_index, batch,
           W_gin1, b_gin1, W_gin2, b_gin2, W_gcn, b_gcn,
           W_f1, b_f1, W_f2, b_f2, W_fl, b_fl,
           W_n2h, b_n2h, W_sp, b_sp, W_hid, b_hid, W_out, b_out):
    src = edge_index[0].astype(_i32)
    dst = edge_index[1].astype(_i32)
    npad = EPAD - E
    srcp = jnp.concatenate([src, jnp.zeros((npad,), _i32)])
    dstp = jnp.concatenate([dst, jnp.full((npad,), NP - 1, _i32)])
    srcs2 = jnp.concatenate([srcp, srcp + NP])

    xp = jnp.concatenate([x, jnp.zeros((NP - N, 128), _f32)])
    pos128 = jnp.zeros((TBLR, 128), _f32).at[:N, :3].set(pos)
    batp3 = jnp.concatenate([batch.astype(_i32),
                             jnp.full((NP - N,), G, _i32)]).reshape(-1, 1, NBLK)

    pk = _bucket_sc(srcp, dstp)

    part = _seg_sum_sc(xp, srcp, dstp, split_edges=True)
    hcat, sch = _tc_gin1(xp, part, W_gin1, b_gin1, W_n2h, b_n2h)

    half = _seg_sum_sc(hcat.reshape(2 * NP, 128), srcs2, dstp,
                       split_edges=False)
    gnn = _tc_gin2pool(hcat, half, W_gin2, b_gin2, batp3)

    relmax = _rel_max_sc(pos128, pk)
    Wsp1 = W_sp[:128]
    Wsp2 = jnp.zeros((16, 128), _f32).at[:3].set(W_sp[128:])
    for i in range(3):
        sch16 = jnp.concatenate([sch, jnp.zeros((TBLR - NP, 128), _f32)])
        maxsch = _seg_max_sc(sch16, pk)
        schout = _tc_spmm(maxsch, relmax, Wsp1, Wsp2, b_sp, batp3,
                          pool=(i == 2))
        if i < 2:
            sch = schout
    sch_out = schout

    Wo128 = jnp.zeros((128, 128), _f32).at[:, :2].set(W_out)
    bo128 = jnp.zeros((1, 128), _f32).at[0, :2].set(b_out)
    out128, gnn_out = _tc_tail(gnn, sch_out, fp, W_gcn, b_gcn, W_f1, b_f1,
                               W_f2, b_f2, W_fl, b_fl, W_hid, b_hid,
                               Wo128, bo128)
    return (out128[:, :2], gnn_out, sch_out)
